# baseline (speedup 1.0000x reference)
import functools

import jax
import jax.numpy as jnp
from jax import lax
from jax.experimental import pallas as pl
from jax.experimental.pallas import tpu as pltpu
from jax.experimental.pallas import tpu_sc as plsc

SEQ_LEN = 2048
D_MODEL = 1024
N_HALF = 4


def _make_sc_broadcast(n: int):
    info = plsc.get_sparse_core_info()
    num_workers = info.num_cores * info.num_subcores
    rows_per_w = SEQ_LEN // num_workers
    half = rows_per_w // N_HALF
    mesh = plsc.VectorSubcoreMesh(core_axis_name="c", subcore_axis_name="s")

    @functools.partial(
        pl.kernel,
        mesh=mesh,
        out_type=jax.ShapeDtypeStruct((SEQ_LEN, n, D_MODEL), jnp.float32),
        scratch_types=(
            [pltpu.VMEM((half, D_MODEL), jnp.float32) for _ in range(N_HALF)]
            + [pltpu.SemaphoreType.DMA, pltpu.SemaphoreType.DMA]
        ),
    )
    def sc_broadcast(pe_hbm, out_hbm, *scratch):
        bufs = list(scratch[:N_HALF])
        sem_in, sem_out = scratch[N_HALF], scratch[N_HALF + 1]
        wid = lax.axis_index("s") * info.num_cores + lax.axis_index("c")
        base = wid * rows_per_w
        reads = [
            pltpu.async_copy(
                pe_hbm.at[pl.ds(base + h * half, half)], bufs[h], sem_in
            )
            for h in range(N_HALF)
        ]
        writes = []
        for h in range(N_HALF):
            reads[h].wait()
            row = base + h * half
            writes.extend(
                pltpu.async_copy(
                    bufs[h], out_hbm.at[pl.ds(row, half), j], sem_out
                )
                for j in range(n)
            )
        for cp in writes:
            cp.wait()

    return sc_broadcast


def kernel(z, pe):
    n = z.shape[1]
    return _make_sc_broadcast(n)(pe)


# speedup vs baseline: 1.1085x; 1.1085x over previous
import functools

import jax
import jax.numpy as jnp
from jax import lax
from jax.experimental import pallas as pl
from jax.experimental.pallas import tpu as pltpu
from jax.experimental.pallas import tpu_sc as plsc

SEQ_LEN = 2048
D_MODEL = 1024
N_HALF = 2


def _make_sc_broadcast(n: int):
    info = plsc.get_sparse_core_info()
    num_workers = info.num_cores * info.num_subcores
    rows_per_w = SEQ_LEN // num_workers
    half = rows_per_w // N_HALF
    mesh = plsc.VectorSubcoreMesh(core_axis_name="c", subcore_axis_name="s")

    @functools.partial(
        pl.kernel,
        mesh=mesh,
        out_type=jax.ShapeDtypeStruct((SEQ_LEN, n, D_MODEL), jnp.float32),
        scratch_types=[
            pltpu.VMEM((half, D_MODEL), jnp.float32),
            pltpu.VMEM((half, D_MODEL), jnp.float32),
            pltpu.SemaphoreType.DMA,
            pltpu.SemaphoreType.DMA,
        ],
    )
    def sc_broadcast(pe_hbm, out_hbm, buf0, buf1, sem_in, sem_out):
        wid = lax.axis_index("s") * info.num_cores + lax.axis_index("c")
        base = wid * rows_per_w
        bufs = [buf0, buf1]
        reads = [
            pltpu.async_copy(
                pe_hbm.at[pl.ds(base + h * half, half)], bufs[h], sem_in
            )
            for h in range(N_HALF)
        ]
        writes = []
        for h in range(N_HALF):
            reads[h].wait()
            row = base + h * half
            writes.extend(
                pltpu.async_copy(
                    bufs[h], out_hbm.at[pl.ds(row, half), j], sem_out
                )
                for j in range(n)
            )
        for cp in writes:
            cp.wait()

    return sc_broadcast


def kernel(z, pe):
    n = z.shape[1]
    return _make_sc_broadcast(n)(pe)
